# Initial kernel scaffold; baseline (speedup 1.0000x reference)
import jax
import jax.numpy as jnp
from jax.experimental import pallas as pl


def kernel(x, edge_index, W1, b1, W2, b2):
    raise NotImplementedError("write your pallas kernel here")


# speedup vs baseline: 24.0980x; 24.0980x over previous
import functools

import jax
import jax.numpy as jnp
from jax import lax
from jax.experimental import pallas as pl
from jax.experimental.pallas import tpu as pltpu
from jax.experimental.pallas import tpu_sc as plsc

N = 10000
E = 320000
D = 128
NC = 2
NS = 16
NW = NC * NS
EPW = E // NW
K = 80
NCHUNK = EPW // K
NPAD = 10240
RPT = NPAD // NS
RZB = 128
PPT = NPAD // NS
GRP = 25
NGRP = NCHUNK // GRP

def _deg_body(dst_hbm, out_hbm, idx_v, ones_v, stage_v, acc_sh):
    cid = lax.axis_index("c")
    sid = lax.axis_index("s")
    wid = cid * NS + sid

    pltpu.sync_copy(dst_hbm.at[wid], idx_v)

    zeros16 = jnp.zeros((16,), jnp.float32)
    for j in range(PPT // 16):
        stage_v[pl.ds(j * 16, 16)] = zeros16
    ones16 = jnp.ones((16,), jnp.float32)
    for j in range(K // 16):
        ones_v[pl.ds(j * 16, 16)] = ones16

    pltpu.sync_copy(stage_v, acc_sh.at[pl.ds(sid * PPT, PPT)])
    plsc.subcore_barrier()

    def body(j, carry):
        pltpu.sync_copy(ones_v, acc_sh.at[idx_v.at[j]], add=True)
        return carry

    lax.fori_loop(0, NCHUNK, body, 0)
    plsc.subcore_barrier()

    pltpu.sync_copy(acc_sh.at[pl.ds(sid * PPT, PPT)], stage_v)
    pltpu.sync_copy(stage_v, out_hbm.at[cid, pl.ds(sid * PPT, PPT)])


def _edge_body(g_hbm, src_hbm, dst_hbm, out_hbm,
               src_g, dst_g, rows_a, rows_b, acc_sh, sem_a, sem_b):
    cid = lax.axis_index("c")
    sid = lax.axis_index("s")
    wid = cid * NS + sid

    zeros16 = jnp.zeros((16,), jnp.float32)

    def zrow(i, carry):
        for jj in range(D // 16):
            rows_a[i, pl.ds(jj * 16, 16)] = zeros16
        return carry

    lax.fori_loop(0, K, zrow, 0)
    base = sid * RPT
    for t in range(RPT // K):
        pltpu.sync_copy(rows_a, acc_sh.at[pl.ds(base + t * K, K)])
    plsc.subcore_barrier()

    def group(g, carry):
        pltpu.sync_copy(src_hbm.at[wid, g], src_g)
        pltpu.sync_copy(dst_hbm.at[wid, g], dst_g)

        def pair(p, c2):
            j = 2 * p
            ca = pltpu.async_copy(g_hbm.at[src_g.at[j]], rows_a, sem_a)
            cb = pltpu.async_copy(g_hbm.at[src_g.at[j + 1]], rows_b, sem_b)
            ca.wait()
            pltpu.sync_copy(rows_a, acc_sh.at[dst_g.at[j]], add=True)
            cb.wait()
            pltpu.sync_copy(rows_b, acc_sh.at[dst_g.at[j + 1]], add=True)
            return c2

        lax.fori_loop(0, GRP // 2, pair, 0)
        if GRP % 2:
            cl = pltpu.async_copy(g_hbm.at[src_g.at[GRP - 1]], rows_a, sem_a)
            cl.wait()
            pltpu.sync_copy(rows_a, acc_sh.at[dst_g.at[GRP - 1]], add=True)
        return carry

    lax.fori_loop(0, NGRP, group, 0)

    plsc.subcore_barrier()
    for t in range(RPT // K):
        sl = pl.ds(base + t * K, K)
        pltpu.sync_copy(acc_sh.at[sl], rows_a)
        pltpu.sync_copy(rows_a, out_hbm.at[cid, sl])


@functools.lru_cache(maxsize=None)
def _sc_kernels():
    mesh = plsc.VectorSubcoreMesh(core_axis_name="c", subcore_axis_name="s",
                                  num_cores=NC, num_subcores=NS)
    deg_kernel = pl.kernel(
        _deg_body,
        out_type=jax.ShapeDtypeStruct((NC, NPAD), jnp.float32),
        mesh=mesh,
        scratch_types=[
            pltpu.VMEM((NCHUNK, K), jnp.int32),
            pltpu.VMEM((K,), jnp.float32),
            pltpu.VMEM((PPT,), jnp.float32),
            pltpu.VMEM_SHARED((NPAD,), jnp.float32),
        ],
    )
    edge_kernel = pl.kernel(
        _edge_body,
        out_type=jax.ShapeDtypeStruct((NC, NPAD, D), jnp.float32),
        mesh=mesh,
        scratch_types=[
            pltpu.VMEM((GRP, K), jnp.int32),
            pltpu.VMEM((GRP, K), jnp.int32),
            pltpu.VMEM((K, D), jnp.float32),
            pltpu.VMEM((K, D), jnp.float32),
            pltpu.VMEM_SHARED((NPAD, D), jnp.float32),
            pltpu.SemaphoreType.DMA,
            pltpu.SemaphoreType.DMA,
        ],
    )
    return deg_kernel, edge_kernel


def _tc_scale_body(degp_ref, x_ref, w_ref, g_ref, dis_ref):
    deg = 1.0 + degp_ref[:, 0:1] + degp_ref[:, 1:2]
    dis = lax.rsqrt(deg)
    h = lax.dot_general(x_ref[...], w_ref[...], (((1,), (0,)), ((), ())),
                        preferred_element_type=jnp.float32)
    g_ref[...] = h * dis
    dis_ref[...] = jnp.broadcast_to(dis, (N, D))


_tc_scale = pl.pallas_call(
    _tc_scale_body,
    out_shape=[
        jax.ShapeDtypeStruct((N, D), jnp.float32),
        jax.ShapeDtypeStruct((N, D), jnp.float32),
    ],
)


def _tc_mid_body(s_ref, g_ref, dis_ref, b_ref, w_ref, g2_ref):
    dis = dis_ref[...]
    u = (s_ref[0, 0:N, :] + s_ref[1, 0:N, :] + g_ref[...]) * dis + b_ref[...]
    r = jnp.maximum(u, 0.0)
    h2 = lax.dot_general(r, w_ref[...], (((1,), (0,)), ((), ())),
                         preferred_element_type=jnp.float32)
    g2_ref[...] = h2 * dis


_tc_mid = pl.pallas_call(
    _tc_mid_body,
    out_shape=jax.ShapeDtypeStruct((N, D), jnp.float32),
)


def _tc_out_body(s_ref, g2_ref, dis_ref, b_ref, out_ref):
    out_ref[...] = (s_ref[0, 0:N, :] + s_ref[1, 0:N, :] + g2_ref[...]) \
        * dis_ref[...] + b_ref[...]


_tc_out = pl.pallas_call(
    _tc_out_body,
    out_shape=jax.ShapeDtypeStruct((N, D), jnp.float32),
)


def kernel(x, edge_index, W1, b1, W2, b2):
    src = edge_index[0].astype(jnp.int32).reshape(NW, NGRP, GRP, K)
    dst = edge_index[1].astype(jnp.int32).reshape(NW, NGRP, GRP, K)
    dstw = edge_index[1].astype(jnp.int32).reshape(NW, NCHUNK, K)
    _deg_kernel, _edge_kernel = _sc_kernels()

    degp = _deg_kernel(dstw)
    degp_t = degp[:, :N].T

    g1, dis2d = _tc_scale(degp_t, x, W1)
    s1 = _edge_kernel(g1, src, dst)
    g2 = _tc_mid(s1, g1, dis2d, b1.reshape(1, D), W2)
    s2 = _edge_kernel(g2, src, dst)
    return _tc_out(s2, g2, dis2d, b2.reshape(1, D))


# speedup vs baseline: 31.6066x; 1.3116x over previous
import functools

import jax
import jax.numpy as jnp
from jax import lax
from jax.experimental import pallas as pl
from jax.experimental.pallas import tpu as pltpu
from jax.experimental.pallas import tpu_sc as plsc

N = 10000
E = 320000
D = 128
NC = 2
NS = 16
NW = NC * NS
EPW = E // NW
KD = 80
NCHUNKD = EPW // KD
GRPD = 25
NGRPD = NCHUNKD // GRPD
K = 40
NCHUNK = EPW // K
NBUF = 5
GRP = 25
NGRP = NCHUNK // GRP
RPG = GRP // NBUF
NPAD = 10240
RPT = NPAD // NS
PPT = NPAD // NS

def _deg_body(dst_hbm, out_hbm, idx_v, ones_v, stage_v, acc_sh):
    cid = lax.axis_index("c")
    sid = lax.axis_index("s")
    wid = cid * NS + sid

    zeros16 = jnp.zeros((16,), jnp.float32)
    for j in range(PPT // 16):
        stage_v[pl.ds(j * 16, 16)] = zeros16
    ones16 = jnp.ones((16,), jnp.float32)
    for j in range(KD // 16):
        ones_v[pl.ds(j * 16, 16)] = ones16

    pltpu.sync_copy(stage_v, acc_sh.at[pl.ds(sid * PPT, PPT)])
    plsc.subcore_barrier()

    def dgroup(g, carry):
        pltpu.sync_copy(dst_hbm.at[wid, g], idx_v)

        def body(j, c2):
            pltpu.sync_copy(ones_v, acc_sh.at[idx_v.at[j]], add=True)
            return c2

        lax.fori_loop(0, GRPD, body, 0)
        return carry

    lax.fori_loop(0, NGRPD, dgroup, 0)
    plsc.subcore_barrier()

    pltpu.sync_copy(acc_sh.at[pl.ds(sid * PPT, PPT)], stage_v)
    pltpu.sync_copy(stage_v, out_hbm.at[cid, pl.ds(sid * PPT, PPT)])


def _edge_body(g_hbm, src_hbm, dst_hbm, out_hbm,
               src_g, dst_g, r0, r1, r2, r3, r4, acc_sh,
               g0, g1, g2, g3, g4, s0, s1, s2, s3, s4, isem):
    rows = (r0, r1, r2, r3, r4)
    gsem = (g0, g1, g2, g3, g4)
    ssem = (s0, s1, s2, s3, s4)
    cid = lax.axis_index("c")
    sid = lax.axis_index("s")
    wid = cid * NS + sid
    base = sid * RPT

    zeros16 = jnp.zeros((16,), jnp.float32)

    def zrow(i, carry):
        for jj in range(D // 16):
            r0[i, pl.ds(jj * 16, 16)] = zeros16
        return carry

    lax.fori_loop(0, K, zrow, 0)
    for t in range(RPT // K):
        pltpu.sync_copy(r0, acc_sh.at[pl.ds(base + t * K, K)])
    plsc.subcore_barrier()

    pltpu.sync_copy(src_hbm.at[wid, 0], src_g.at[0])
    pltpu.sync_copy(dst_hbm.at[wid, 0], dst_g.at[0])
    pltpu.async_copy(src_hbm.at[wid, 1], src_g.at[1], isem)
    pltpu.async_copy(dst_hbm.at[wid, 1], dst_g.at[1], isem)
    for sl in range(NBUF):
        pltpu.async_copy(g_hbm.at[src_g.at[0, sl]], rows[sl], gsem[sl])

    def ground(lr, g):
        p = g % 2
        pn = 1 - p
        last_g = g == NGRP - 1

        @pl.when(jnp.logical_and(lr == 1, jnp.logical_not(last_g)))
        def _():
            pltpu.async_copy(src_hbm.at[wid, g + 1], src_g.at[pn], isem)
            pltpu.async_copy(dst_hbm.at[wid, g + 1], dst_g.at[pn], isem)

        @pl.when(jnp.logical_and(lr == RPG - 1, jnp.logical_not(last_g)))
        def _():
            pltpu.make_async_copy(src_hbm.at[wid, g + 1], src_g.at[pn],
                                  isem).wait()
            pltpu.make_async_copy(dst_hbm.at[wid, g + 1], dst_g.at[pn],
                                  isem).wait()

        for sl in range(NBUF):
            lc = lr * NBUF + sl
            pltpu.make_async_copy(g_hbm.at[src_g.at[p, lc]], rows[sl],
                                  gsem[sl]).wait()
            pltpu.async_copy(rows[sl], acc_sh.at[dst_g.at[p, lc]],
                             ssem[sl], add=True)

        in_group = lr < RPG - 1
        to_next_group = jnp.logical_and(lr == RPG - 1,
                                        jnp.logical_not(last_g))
        for sl in range(NBUF):
            lc = lr * NBUF + sl

            @pl.when(jnp.logical_or(in_group, to_next_group))
            def _():
                pltpu.make_async_copy(rows[sl], acc_sh.at[dst_g.at[p, lc]],
                                      ssem[sl]).wait()

            @pl.when(in_group)
            def _():
                nlc = (lr + 1) * NBUF + sl
                pltpu.async_copy(g_hbm.at[src_g.at[p, nlc]], rows[sl],
                                 gsem[sl])

            @pl.when(to_next_group)
            def _():
                pltpu.async_copy(g_hbm.at[src_g.at[pn, sl]], rows[sl],
                                 gsem[sl])

    def group(g, carry):
        def rbody(lr, c2):
            ground(lr, g)
            return c2
        lax.fori_loop(0, RPG, rbody, 0)
        return carry

    lax.fori_loop(0, NGRP, group, 0)

    pl_last = (NGRP - 1) % 2
    for sl in range(NBUF):
        lc = (RPG - 1) * NBUF + sl
        pltpu.make_async_copy(rows[sl], acc_sh.at[dst_g.at[pl_last, lc]],
                              ssem[sl]).wait()

    plsc.subcore_barrier()
    for t in range(RPT // K):
        sp = pl.ds(base + t * K, K)
        pltpu.sync_copy(acc_sh.at[sp], r0)
        pltpu.sync_copy(r0, out_hbm.at[cid, sp])


@functools.lru_cache(maxsize=None)
def _sc_kernels():
    mesh = plsc.VectorSubcoreMesh(core_axis_name="c", subcore_axis_name="s",
                                  num_cores=NC, num_subcores=NS)
    deg_kernel = pl.kernel(
        _deg_body,
        out_type=jax.ShapeDtypeStruct((NC, NPAD), jnp.float32),
        mesh=mesh,
        scratch_types=[
            pltpu.VMEM((GRPD, KD), jnp.int32),
            pltpu.VMEM((KD,), jnp.float32),
            pltpu.VMEM((PPT,), jnp.float32),
            pltpu.VMEM_SHARED((NPAD,), jnp.float32),
        ],
    )
    edge_kernel = pl.kernel(
        _edge_body,
        out_type=jax.ShapeDtypeStruct((NC, NPAD, D), jnp.float32),
        mesh=mesh,
        scratch_types=(
            [pltpu.VMEM((2, GRP, K), jnp.int32)] * 2
            + [pltpu.VMEM((K, D), jnp.float32)] * NBUF
            + [pltpu.VMEM_SHARED((NPAD, D), jnp.float32)]
            + [pltpu.SemaphoreType.DMA] * (2 * NBUF + 1)
        ),
    )
    return deg_kernel, edge_kernel


def _tc_scale_body(degp_ref, x_ref, w_ref, g_ref, dis_ref):
    deg = 1.0 + degp_ref[:, 0:1] + degp_ref[:, 1:2]
    dis = lax.rsqrt(deg)
    h = lax.dot_general(x_ref[...], w_ref[...], (((1,), (0,)), ((), ())),
                        preferred_element_type=jnp.float32)
    g_ref[...] = h * dis
    dis_ref[...] = jnp.broadcast_to(dis, (N, D))


_tc_scale = pl.pallas_call(
    _tc_scale_body,
    out_shape=[
        jax.ShapeDtypeStruct((N, D), jnp.float32),
        jax.ShapeDtypeStruct((N, D), jnp.float32),
    ],
)


def _tc_mid_body(s_ref, g_ref, dis_ref, b_ref, w_ref, g2_ref):
    dis = dis_ref[...]
    u = (s_ref[0, 0:N, :] + s_ref[1, 0:N, :] + g_ref[...]) * dis + b_ref[...]
    r = jnp.maximum(u, 0.0)
    h2 = lax.dot_general(r, w_ref[...], (((1,), (0,)), ((), ())),
                         preferred_element_type=jnp.float32)
    g2_ref[...] = h2 * dis


_tc_mid = pl.pallas_call(
    _tc_mid_body,
    out_shape=jax.ShapeDtypeStruct((N, D), jnp.float32),
)


def _tc_out_body(s_ref, g2_ref, dis_ref, b_ref, out_ref):
    out_ref[...] = (s_ref[0, 0:N, :] + s_ref[1, 0:N, :] + g2_ref[...]) \
        * dis_ref[...] + b_ref[...]


_tc_out = pl.pallas_call(
    _tc_out_body,
    out_shape=jax.ShapeDtypeStruct((N, D), jnp.float32),
)


def kernel(x, edge_index, W1, b1, W2, b2):
    src = edge_index[0].astype(jnp.int32).reshape(NW, NGRP, GRP, K)
    dst = edge_index[1].astype(jnp.int32).reshape(NW, NGRP, GRP, K)
    dstw = edge_index[1].astype(jnp.int32).reshape(NW, NGRPD, GRPD, KD)
    _deg_kernel, _edge_kernel = _sc_kernels()

    degp = _deg_kernel(dstw)
    degp_t = degp[:, :N].T

    g1, dis2d = _tc_scale(degp_t, x, W1)
    s1 = _edge_kernel(g1, src, dst)
    g2 = _tc_mid(s1, g1, dis2d, b1.reshape(1, D), W2)
    s2 = _edge_kernel(g2, src, dst)
    return _tc_out(s2, g2, dis2d, b2.reshape(1, D))


# speedup vs baseline: 33.0643x; 1.0461x over previous
import functools

import jax
import jax.numpy as jnp
from jax import lax
from jax.experimental import pallas as pl
from jax.experimental.pallas import tpu as pltpu
from jax.experimental.pallas import tpu_sc as plsc

N = 10000
E = 320000
D = 128
NC = 2
NS = 16
NW = NC * NS
EPW = E // NW
KD = 80
NCHUNKD = EPW // KD
GRPD = 25
NGRPD = NCHUNKD // GRPD
K = 40
NCHUNK = EPW // K
NBUF = 5
GRP = 25
NGRP = NCHUNK // GRP
RPG = GRP // NBUF
NPAD = 10240
RPT = NPAD // NS
PPT = NPAD // NS

def _deg_body(dst_hbm, out_hbm, idx_v, ones_v, stage_v, acc_sh, dsem):
    cid = lax.axis_index("c")
    sid = lax.axis_index("s")
    wid = cid * NS + sid

    zeros16 = jnp.zeros((16,), jnp.float32)
    for j in range(PPT // 16):
        stage_v[pl.ds(j * 16, 16)] = zeros16
    ones16 = jnp.ones((16,), jnp.float32)
    for j in range(KD // 16):
        ones_v[pl.ds(j * 16, 16)] = ones16

    pltpu.sync_copy(stage_v, acc_sh.at[pl.ds(sid * PPT, PPT)])
    plsc.subcore_barrier()

    def dgroup(g, carry):
        pltpu.sync_copy(dst_hbm.at[wid, g], idx_v)

        def fire(j, c2):
            pltpu.async_copy(ones_v, acc_sh.at[idx_v.at[j]], dsem, add=True)
            return c2

        lax.fori_loop(0, GRPD, fire, 0)

        def drain(j, c2):
            pltpu.make_async_copy(ones_v, acc_sh.at[idx_v.at[j]],
                                  dsem).wait()
            return c2

        lax.fori_loop(0, GRPD, drain, 0)
        return carry

    lax.fori_loop(0, NGRPD, dgroup, 0)
    plsc.subcore_barrier()

    pltpu.sync_copy(acc_sh.at[pl.ds(sid * PPT, PPT)],
                    out_hbm.at[cid, pl.ds(sid * PPT, PPT)])


def _edge_body(g_hbm, src_hbm, dst_hbm, out_hbm,
               src_g, dst_g, r0, r1, r2, r3, r4, zb, acc_sh,
               g0, g1, g2, g3, g4, s0, s1, s2, s3, s4, isem):
    rows = (r0, r1, r2, r3, r4)
    gsem = (g0, g1, g2, g3, g4)
    ssem = (s0, s1, s2, s3, s4)
    cid = lax.axis_index("c")
    sid = lax.axis_index("s")
    wid = cid * NS + sid
    base = sid * RPT

    pltpu.sync_copy(src_hbm.at[wid, 0], src_g.at[0])
    pltpu.sync_copy(dst_hbm.at[wid, 0], dst_g.at[0])
    pltpu.async_copy(src_hbm.at[wid, 1], src_g.at[1], isem)
    pltpu.async_copy(dst_hbm.at[wid, 1], dst_g.at[1], isem)
    for sl in range(NBUF):
        pltpu.async_copy(g_hbm.at[src_g.at[0, sl]], rows[sl], gsem[sl])

    zeros16 = jnp.zeros((16,), jnp.float32)
    ZR = 32

    def zrow(i, carry):
        for jj in range(D // 16):
            zb[i, pl.ds(jj * 16, 16)] = zeros16
        return carry

    lax.fori_loop(0, ZR, zrow, 0)
    for t in range(RPT // ZR):
        pltpu.sync_copy(zb, acc_sh.at[pl.ds(base + t * ZR, ZR)])
    plsc.subcore_barrier()

    def ground(lr, g):
        p = g % 2
        pn = 1 - p
        last_g = g == NGRP - 1

        @pl.when(jnp.logical_and(lr == 1, jnp.logical_not(last_g)))
        def _():
            pltpu.async_copy(src_hbm.at[wid, g + 1], src_g.at[pn], isem)
            pltpu.async_copy(dst_hbm.at[wid, g + 1], dst_g.at[pn], isem)

        @pl.when(jnp.logical_and(lr == RPG - 1, jnp.logical_not(last_g)))
        def _():
            pltpu.make_async_copy(src_hbm.at[wid, g + 1], src_g.at[pn],
                                  isem).wait()
            pltpu.make_async_copy(dst_hbm.at[wid, g + 1], dst_g.at[pn],
                                  isem).wait()

        for sl in range(NBUF):
            lc = lr * NBUF + sl
            pltpu.make_async_copy(g_hbm.at[src_g.at[p, lc]], rows[sl],
                                  gsem[sl]).wait()
            pltpu.async_copy(rows[sl], acc_sh.at[dst_g.at[p, lc]],
                             ssem[sl], add=True)

        in_group = lr < RPG - 1
        to_next_group = jnp.logical_and(lr == RPG - 1,
                                        jnp.logical_not(last_g))
        for sl in range(NBUF):
            lc = lr * NBUF + sl

            @pl.when(jnp.logical_or(in_group, to_next_group))
            def _():
                pltpu.make_async_copy(rows[sl], acc_sh.at[dst_g.at[p, lc]],
                                      ssem[sl]).wait()

            @pl.when(in_group)
            def _():
                nlc = (lr + 1) * NBUF + sl
                pltpu.async_copy(g_hbm.at[src_g.at[p, nlc]], rows[sl],
                                 gsem[sl])

            @pl.when(to_next_group)
            def _():
                pltpu.async_copy(g_hbm.at[src_g.at[pn, sl]], rows[sl],
                                 gsem[sl])

    def group(g, carry):
        def rbody(lr, c2):
            ground(lr, g)
            return c2
        lax.fori_loop(0, RPG, rbody, 0)
        return carry

    lax.fori_loop(0, NGRP, group, 0)

    pl_last = (NGRP - 1) % 2
    for sl in range(NBUF):
        lc = (RPG - 1) * NBUF + sl
        pltpu.make_async_copy(rows[sl], acc_sh.at[dst_g.at[pl_last, lc]],
                              ssem[sl]).wait()

    plsc.subcore_barrier()
    sp = pl.ds(base, RPT)
    pltpu.sync_copy(acc_sh.at[sp], out_hbm.at[cid, sp])


@functools.lru_cache(maxsize=None)
def _sc_kernels():
    mesh = plsc.VectorSubcoreMesh(core_axis_name="c", subcore_axis_name="s",
                                  num_cores=NC, num_subcores=NS)
    deg_kernel = pl.kernel(
        _deg_body,
        out_type=jax.ShapeDtypeStruct((NC, NPAD), jnp.float32),
        mesh=mesh,
        scratch_types=[
            pltpu.VMEM((GRPD, KD), jnp.int32),
            pltpu.VMEM((KD,), jnp.float32),
            pltpu.VMEM((PPT,), jnp.float32),
            pltpu.VMEM_SHARED((NPAD,), jnp.float32),
            pltpu.SemaphoreType.DMA,
        ],
    )
    edge_kernel = pl.kernel(
        _edge_body,
        out_type=jax.ShapeDtypeStruct((NC, NPAD, D), jnp.float32),
        mesh=mesh,
        scratch_types=(
            [pltpu.VMEM((2, GRP, K), jnp.int32)] * 2
            + [pltpu.VMEM((K, D), jnp.float32)] * NBUF
            + [pltpu.VMEM((32, D), jnp.float32)]
            + [pltpu.VMEM_SHARED((NPAD, D), jnp.float32)]
            + [pltpu.SemaphoreType.DMA] * (2 * NBUF + 1)
        ),
    )
    return deg_kernel, edge_kernel


def _tc_scale_body(degp_ref, x_ref, w_ref, g_ref, dis_ref):
    deg = 1.0 + degp_ref[:, 0:1] + degp_ref[:, 1:2]
    dis = lax.rsqrt(deg)
    h = lax.dot_general(x_ref[...], w_ref[...], (((1,), (0,)), ((), ())),
                        preferred_element_type=jnp.float32)
    g_ref[...] = h * dis
    dis_ref[...] = jnp.broadcast_to(dis, (N, D))


_tc_scale = pl.pallas_call(
    _tc_scale_body,
    out_shape=[
        jax.ShapeDtypeStruct((N, D), jnp.float32),
        jax.ShapeDtypeStruct((N, D), jnp.float32),
    ],
)


def _tc_mid_body(s_ref, g_ref, dis_ref, b_ref, w_ref, g2_ref):
    dis = dis_ref[...]
    u = (s_ref[0, 0:N, :] + s_ref[1, 0:N, :] + g_ref[...]) * dis + b_ref[...]
    r = jnp.maximum(u, 0.0)
    h2 = lax.dot_general(r, w_ref[...], (((1,), (0,)), ((), ())),
                         preferred_element_type=jnp.float32)
    g2_ref[...] = h2 * dis


_tc_mid = pl.pallas_call(
    _tc_mid_body,
    out_shape=jax.ShapeDtypeStruct((N, D), jnp.float32),
)


def _tc_out_body(s_ref, g2_ref, dis_ref, b_ref, out_ref):
    out_ref[...] = (s_ref[0, 0:N, :] + s_ref[1, 0:N, :] + g2_ref[...]) \
        * dis_ref[...] + b_ref[...]


_tc_out = pl.pallas_call(
    _tc_out_body,
    out_shape=jax.ShapeDtypeStruct((N, D), jnp.float32),
)


def kernel(x, edge_index, W1, b1, W2, b2):
    src = edge_index[0].astype(jnp.int32).reshape(NW, NGRP, GRP, K)
    dst = edge_index[1].astype(jnp.int32).reshape(NW, NGRP, GRP, K)
    dstw = edge_index[1].astype(jnp.int32).reshape(NW, NGRPD, GRPD, KD)
    _deg_kernel, _edge_kernel = _sc_kernels()

    degp = _deg_kernel(dstw)
    degp_t = degp[:, :N].T

    g1, dis2d = _tc_scale(degp_t, x, W1)
    s1 = _edge_kernel(g1, src, dst)
    g2 = _tc_mid(s1, g1, dis2d, b1.reshape(1, D), W2)
    s2 = _edge_kernel(g2, src, dst)
    return _tc_out(s2, g2, dis2d, b2.reshape(1, D))


# speedup vs baseline: 33.0848x; 1.0006x over previous
import functools

import jax
import jax.numpy as jnp
from jax import lax
from jax.experimental import pallas as pl
from jax.experimental.pallas import tpu as pltpu
from jax.experimental.pallas import tpu_sc as plsc

N = 10000
E = 320000
D = 128
NC = 2
NS = 16
NW = NC * NS
EPW = E // NW
KD = 80
NCHUNKD = EPW // KD
GRPD = 25
NGRPD = NCHUNKD // GRPD
K = 40
NCHUNK = EPW // K
NBUF = 5
GRP = 25
NGRP = NCHUNK // GRP
RPG = GRP // NBUF
NPAD = 10240
RPT = NPAD // NS
PPT = NPAD // NS

def _deg_body(ei_hbm, out_hbm, idx_v, ones_v, stage_v, acc_sh, dsem):
    cid = lax.axis_index("c")
    sid = lax.axis_index("s")
    wid = cid * NS + sid

    zeros16 = jnp.zeros((16,), jnp.float32)
    for j in range(PPT // 16):
        stage_v[pl.ds(j * 16, 16)] = zeros16
    ones16 = jnp.ones((16,), jnp.float32)
    for off in (0, 16, 24):
        ones_v[pl.ds(off, 16)] = ones16

    pltpu.sync_copy(stage_v, acc_sh.at[pl.ds(sid * PPT, PPT)])
    plsc.subcore_barrier()

    def dgroup(g, carry):
        pltpu.sync_copy(ei_hbm.at[1, wid, g], idx_v)

        def fire(j, c2):
            pltpu.async_copy(ones_v, acc_sh.at[idx_v.at[j]], dsem, add=True)
            return c2

        lax.fori_loop(0, GRP, fire, 0)

        def drain(j, c2):
            pltpu.make_async_copy(ones_v, acc_sh.at[idx_v.at[j]],
                                  dsem).wait()
            return c2

        lax.fori_loop(0, GRP, drain, 0)
        return carry

    lax.fori_loop(0, NGRP, dgroup, 0)
    plsc.subcore_barrier()

    pltpu.sync_copy(acc_sh.at[pl.ds(sid * PPT, PPT)],
                    out_hbm.at[cid, pl.ds(sid * PPT, PPT)])


def _edge_body(g_hbm, ei_hbm, out_hbm,
               src_g, dst_g, r0, r1, r2, r3, r4, zb, acc_sh,
               g0, g1, g2, g3, g4, s0, s1, s2, s3, s4, isem):
    rows = (r0, r1, r2, r3, r4)
    gsem = (g0, g1, g2, g3, g4)
    ssem = (s0, s1, s2, s3, s4)
    cid = lax.axis_index("c")
    sid = lax.axis_index("s")
    wid = cid * NS + sid
    base = sid * RPT

    pltpu.sync_copy(ei_hbm.at[0, wid, 0], src_g.at[0])
    pltpu.sync_copy(ei_hbm.at[1, wid, 0], dst_g.at[0])
    pltpu.async_copy(ei_hbm.at[0, wid, 1], src_g.at[1], isem)
    pltpu.async_copy(ei_hbm.at[1, wid, 1], dst_g.at[1], isem)
    for sl in range(NBUF):
        pltpu.async_copy(g_hbm.at[src_g.at[0, sl]], rows[sl], gsem[sl])

    zeros16 = jnp.zeros((16,), jnp.float32)
    ZR = 32

    def zrow(i, carry):
        for jj in range(D // 16):
            zb[i, pl.ds(jj * 16, 16)] = zeros16
        return carry

    lax.fori_loop(0, ZR, zrow, 0)
    for t in range(RPT // ZR):
        pltpu.sync_copy(zb, acc_sh.at[pl.ds(base + t * ZR, ZR)])
    plsc.subcore_barrier()

    def ground(lr, g):
        p = g % 2
        pn = 1 - p
        last_g = g == NGRP - 1

        @pl.when(jnp.logical_and(lr == 1, jnp.logical_not(last_g)))
        def _():
            pltpu.async_copy(ei_hbm.at[0, wid, g + 1], src_g.at[pn], isem)
            pltpu.async_copy(ei_hbm.at[1, wid, g + 1], dst_g.at[pn], isem)

        @pl.when(jnp.logical_and(lr == RPG - 1, jnp.logical_not(last_g)))
        def _():
            pltpu.make_async_copy(ei_hbm.at[0, wid, g + 1], src_g.at[pn],
                                  isem).wait()
            pltpu.make_async_copy(ei_hbm.at[1, wid, g + 1], dst_g.at[pn],
                                  isem).wait()

        for sl in range(NBUF):
            lc = lr * NBUF + sl
            pltpu.make_async_copy(g_hbm.at[src_g.at[p, lc]], rows[sl],
                                  gsem[sl]).wait()
            pltpu.async_copy(rows[sl], acc_sh.at[dst_g.at[p, lc]],
                             ssem[sl], add=True)

        in_group = lr < RPG - 1
        to_next_group = jnp.logical_and(lr == RPG - 1,
                                        jnp.logical_not(last_g))
        for sl in range(NBUF):
            lc = lr * NBUF + sl

            @pl.when(jnp.logical_or(in_group, to_next_group))
            def _():
                pltpu.make_async_copy(rows[sl], acc_sh.at[dst_g.at[p, lc]],
                                      ssem[sl]).wait()

            @pl.when(in_group)
            def _():
                nlc = (lr + 1) * NBUF + sl
                pltpu.async_copy(g_hbm.at[src_g.at[p, nlc]], rows[sl],
                                 gsem[sl])

            @pl.when(to_next_group)
            def _():
                pltpu.async_copy(g_hbm.at[src_g.at[pn, sl]], rows[sl],
                                 gsem[sl])

    def group(g, carry):
        def rbody(lr, c2):
            ground(lr, g)
            return c2
        lax.fori_loop(0, RPG, rbody, 0)
        return carry

    lax.fori_loop(0, NGRP, group, 0)

    pl_last = (NGRP - 1) % 2
    for sl in range(NBUF):
        lc = (RPG - 1) * NBUF + sl
        pltpu.make_async_copy(rows[sl], acc_sh.at[dst_g.at[pl_last, lc]],
                              ssem[sl]).wait()

    plsc.subcore_barrier()
    sp = pl.ds(base, RPT)
    pltpu.sync_copy(acc_sh.at[sp], out_hbm.at[cid, sp])


@functools.lru_cache(maxsize=None)
def _sc_kernels():
    mesh = plsc.VectorSubcoreMesh(core_axis_name="c", subcore_axis_name="s",
                                  num_cores=NC, num_subcores=NS)
    deg_kernel = pl.kernel(
        _deg_body,
        out_type=jax.ShapeDtypeStruct((NC, NPAD), jnp.float32),
        mesh=mesh,
        scratch_types=[
            pltpu.VMEM((GRP, K), jnp.int32),
            pltpu.VMEM((K,), jnp.float32),
            pltpu.VMEM((PPT,), jnp.float32),
            pltpu.VMEM_SHARED((NPAD,), jnp.float32),
            pltpu.SemaphoreType.DMA,
        ],
    )
    edge_kernel = pl.kernel(
        _edge_body,
        out_type=jax.ShapeDtypeStruct((NC, NPAD, D), jnp.float32),
        mesh=mesh,
        scratch_types=(
            [pltpu.VMEM((2, GRP, K), jnp.int32)] * 2
            + [pltpu.VMEM((K, D), jnp.float32)] * NBUF
            + [pltpu.VMEM((32, D), jnp.float32)]
            + [pltpu.VMEM_SHARED((NPAD, D), jnp.float32)]
            + [pltpu.SemaphoreType.DMA] * (2 * NBUF + 1)
        ),
    )
    return deg_kernel, edge_kernel


def _dis(degp_ref):
    deg = 1.0 + degp_ref[:, 0:1] + degp_ref[:, 1:2]
    return lax.rsqrt(deg)


def _tc_scale_body(degp_ref, x_ref, w_ref, g_ref):
    h = lax.dot_general(x_ref[...], w_ref[...], (((1,), (0,)), ((), ())),
                        preferred_element_type=jnp.float32)
    g_ref[...] = h * _dis(degp_ref)


_tc_scale = pl.pallas_call(
    _tc_scale_body,
    out_shape=jax.ShapeDtypeStruct((N, D), jnp.float32),
)


def _tc_mid_body(s_ref, g_ref, degp_ref, b_ref, w_ref, g2_ref):
    dis = _dis(degp_ref)
    u = (s_ref[0, 0:N, :] + s_ref[1, 0:N, :] + g_ref[...]) * dis + b_ref[...]
    r = jnp.maximum(u, 0.0)
    h2 = lax.dot_general(r, w_ref[...], (((1,), (0,)), ((), ())),
                         preferred_element_type=jnp.float32)
    g2_ref[...] = h2 * dis


_tc_mid = pl.pallas_call(
    _tc_mid_body,
    out_shape=jax.ShapeDtypeStruct((N, D), jnp.float32),
)


def _tc_out_body(s_ref, g2_ref, degp_ref, b_ref, out_ref):
    out_ref[...] = (s_ref[0, 0:N, :] + s_ref[1, 0:N, :] + g2_ref[...]) \
        * _dis(degp_ref) + b_ref[...]


_tc_out = pl.pallas_call(
    _tc_out_body,
    out_shape=jax.ShapeDtypeStruct((N, D), jnp.float32),
)


def kernel(x, edge_index, W1, b1, W2, b2):
    ei = edge_index.astype(jnp.int32).reshape(2, NW, NGRP, GRP, K)
    _deg_kernel, _edge_kernel = _sc_kernels()

    degp = _deg_kernel(ei)
    degp_t = degp[:, :N].T

    g1 = _tc_scale(degp_t, x, W1)
    s1 = _edge_kernel(g1, ei)
    g2 = _tc_mid(s1, g1, degp_t, b1.reshape(1, D), W2)
    s2 = _edge_kernel(g2, ei)
    return _tc_out(s2, g2, degp_t, b2.reshape(1, D))
